# baseline (speedup 1.0000x reference)
import functools

import jax
import jax.numpy as jnp
from jax import lax
from jax.experimental import pallas as pl
from jax.experimental.pallas import tpu as pltpu
from jax.experimental.pallas import tpu_sc as plsc

_B = 16384
_D = 32
_NC, _NS = 2, 16
_NW = _NC * _NS
_BPW = _B // _NW
_CHUNK = 128
_NCHUNK = _BPW // _CHUNK


def _sc_gather4(user, item, umf, imf, umlp, imlp):
    mesh = plsc.VectorSubcoreMesh(core_axis_name="c", subcore_axis_name="s")
    out = jax.ShapeDtypeStruct((_B, _D), jnp.float32)
    scratch = [
        pltpu.VMEM((_NCHUNK, _CHUNK), jnp.int32),
        pltpu.VMEM((_NCHUNK, _CHUNK), jnp.int32),
    ]
    scratch += [pltpu.VMEM((_CHUNK, _D), jnp.float32) for _ in range(4 * _NCHUNK)]
    scratch += [pltpu.SemaphoreType.DMA for _ in range(4 * _NCHUNK)]

    @functools.partial(pl.kernel, out_type=(out, out, out, out), mesh=mesh,
                       scratch_types=scratch,
                       compiler_params=pltpu.CompilerParams(
                           use_tc_tiling_on_sc=False))
    def k(user_h, item_h, umf_h, imf_h, umlp_h, imlp_h,
          o_umf, o_imf, o_umlp, o_imlp, *scr):
        uidx, iidx = scr[0], scr[1]
        bufs = scr[2:2 + 4 * _NCHUNK]
        sems = scr[2 + 4 * _NCHUNK:]
        wid = lax.axis_index("s") * _NC + lax.axis_index("c")
        base = wid * _BPW
        for c in range(_NCHUNK):
            pltpu.sync_copy(user_h.at[pl.ds(base + c * _CHUNK, _CHUNK)], uidx.at[c])
            pltpu.sync_copy(item_h.at[pl.ds(base + c * _CHUNK, _CHUNK)], iidx.at[c])
        tables = (umf_h, imf_h, umlp_h, imlp_h)
        idxs = (uidx, iidx, uidx, iidx)
        copies = []
        for c in range(_NCHUNK):
            for t in range(4):
                j = c * 4 + t
                copies.append(
                    pltpu.async_copy(tables[t].at[idxs[t].at[c]], bufs[j], sems[j]))
        outs = (o_umf, o_imf, o_umlp, o_imlp)
        for c in range(_NCHUNK):
            for t in range(4):
                j = c * 4 + t
                copies[j].wait()
                pltpu.sync_copy(bufs[j], outs[t].at[pl.ds(base + c * _CHUNK, _CHUNK)])

    return k(user, item, umf, imf, umlp, imlp)


_BLK = 2048


def _dense_body(umf_ref, imf_ref, umlp_ref, imlp_ref,
                w1_ref, b1_ref, w2_ref, b2_ref, wp_ref, bp_ref, o_ref):
    x = jnp.concatenate([umlp_ref[...], imlp_ref[...]], axis=1)
    h = lax.dot_general(x, w1_ref[...], (((1,), (1,)), ((), ())),
                        preferred_element_type=jnp.float32)
    h = jnp.maximum(h + b1_ref[...], 0.0)
    h = lax.dot_general(h, w2_ref[...], (((1,), (1,)), ((), ())),
                        preferred_element_type=jnp.float32)
    h = jnp.maximum(h + b2_ref[...], 0.0)
    mf = umf_ref[...] * imf_ref[...]
    s = (jnp.sum(mf * wp_ref[:, :_D], axis=1, keepdims=True)
         + jnp.sum(h * wp_ref[:, _D:], axis=1, keepdims=True)
         + bp_ref[...])
    o_ref[...] = s


def _tc_dense(umf_g, imf_g, umlp_g, imlp_g, W1, b1, W2, b2, Wp, bp):
    grid = _B // _BLK
    row = lambda i: (i, 0)
    fixed2 = lambda i: (0, 0)
    out = pl.pallas_call(
        _dense_body,
        grid=(grid,),
        in_specs=[
            pl.BlockSpec((_BLK, _D), row),
            pl.BlockSpec((_BLK, _D), row),
            pl.BlockSpec((_BLK, _D), row),
            pl.BlockSpec((_BLK, _D), row),
            pl.BlockSpec((2 * _D, 2 * _D), fixed2),
            pl.BlockSpec((1, 2 * _D), fixed2),
            pl.BlockSpec((_D, 2 * _D), fixed2),
            pl.BlockSpec((1, _D), fixed2),
            pl.BlockSpec((1, 2 * _D), fixed2),
            pl.BlockSpec((1, 1), fixed2),
        ],
        out_specs=pl.BlockSpec((_BLK, 1), row),
        out_shape=jax.ShapeDtypeStruct((_B, 1), jnp.float32),
    )(umf_g, imf_g, umlp_g, imlp_g, W1, b1.reshape(1, 2 * _D), W2,
      b2.reshape(1, _D), Wp, bp.reshape(1, 1))
    return out[:, 0]


def kernel(user, item, user_mf, item_mf, user_mlp, item_mlp,
           W1, b1, W2, b2, Wp, bp):
    umf_g, imf_g, umlp_g, imlp_g = _sc_gather4(
        user, item, user_mf, item_mf, user_mlp, item_mlp)
    return _tc_dense(umf_g, imf_g, umlp_g, imlp_g, W1, b1, W2, b2, Wp, bp)


# speedup vs baseline: 3.7467x; 3.7467x over previous
import functools

import jax
import jax.numpy as jnp
from jax import lax
from jax.experimental import pallas as pl
from jax.experimental.pallas import tpu as pltpu
from jax.experimental.pallas import tpu_sc as plsc

_B = 16384
_D = 32
_V = 1000000
_WLAST = _V // 128 - 1
_TAIL0 = (_WLAST + 1) * 128
_NC, _NS = 2, 16
_NW = _NC * _NS
_BPW = _B // _NW
_DEPTH = 4
_HALF = _BPW // 2


def _sc_gather4(user, item, t0, t1, t2, t3):
    mesh = plsc.VectorSubcoreMesh(core_axis_name="c", subcore_axis_name="s")
    out = jax.ShapeDtypeStruct((_D, _B), jnp.float32)
    scratch = [
        pltpu.VMEM((_BPW,), jnp.int32),
        pltpu.VMEM((_BPW,), jnp.int32),
    ]
    scratch += [pltpu.VMEM((_D, 128), jnp.float32) for _ in range(4 * _DEPTH)]
    scratch += [pltpu.VMEM((_D, _HALF), jnp.float32) for _ in range(4)]
    scratch += [pltpu.VMEM((_D, 64), jnp.float32) for _ in range(4)]
    scratch += [pltpu.SemaphoreType.DMA for _ in range(4 * _DEPTH)]

    @functools.partial(pl.kernel, out_type=(out, out, out, out), mesh=mesh,
                       scratch_types=scratch,
                       compiler_params=pltpu.CompilerParams(
                           needs_layout_passes=False))
    def k(user_h, item_h, h0, h1, h2, h3, o0, o1, o2, o3, *scr):
        uidx, iidx = scr[0], scr[1]
        bufs = scr[2:2 + 4 * _DEPTH]
        stage = scr[2 + 4 * _DEPTH:6 + 4 * _DEPTH]
        tails = scr[6 + 4 * _DEPTH:10 + 4 * _DEPTH]
        sems = scr[10 + 4 * _DEPTH:]
        tables = (h0, h1, h2, h3)
        outs = (o0, o1, o2, o3)
        wid = lax.axis_index("s") * _NC + lax.axis_index("c")
        base = wid * _BPW
        pltpu.sync_copy(user_h.at[pl.ds(base, _BPW)], uidx)
        pltpu.sync_copy(item_h.at[pl.ds(base, _BPW)], iidx)
        for t in range(4):
            pltpu.sync_copy(tables[t].at[:, pl.ds(_TAIL0, 64)], tails[t])

        def load_vecs(j):
            ru = uidx[pl.ds(j, 16)]
            ri = iidx[pl.ds(j, 16)]
            out_v = []
            for r in (ru, ri):
                start = jnp.minimum(lax.shift_right_logical(r, 7),
                                    _WLAST) * 128
                out_v.append((start, r - start))
            return out_v

        def fetch(vecs, s, slot):
            for t in range(4):
                start, _ = vecs[t % 2]
                st = pl.multiple_of(start[s], 128)
                for q in range(4):
                    pltpu.async_copy(
                        tables[t].at[pl.ds(8 * q, 8), pl.ds(st, 128)],
                        bufs[slot * 4 + t].at[pl.ds(8 * q, 8), :],
                        sems[slot * 4 + t])

        def extract(vecs, s, slot, col0):
            it = lax.iota(jnp.int32, 16)
            col = jnp.full((16,), col0 + s, dtype=jnp.int32)
            for t in range(4):
                _, lane = vecs[t % 2]
                ln = lane[s]
                lane_v = jnp.full((16,), jnp.minimum(ln, 127), dtype=jnp.int32)
                tlane_v = jnp.full((16,), jnp.clip(ln - 128, 0, 63),
                                   dtype=jnp.int32)
                is_tail = jnp.full((16,), ln, dtype=jnp.int32) >= 128
                b = bufs[slot * 4 + t]
                pltpu.make_async_copy(tables[t].at[:, pl.ds(0, 128)], b,
                                      sems[slot * 4 + t]).wait()
                for h in range(2):
                    rows = it + h * 16
                    v = plsc.load_gather(b, [rows, lane_v])
                    vt = plsc.load_gather(tails[t], [rows, tlane_v])
                    v = jnp.where(is_tail, vt, v)
                    plsc.store_scatter(stage[t], [rows, col], v)

        for half in range(2):
            j0 = half * _HALF
            vecs0 = load_vecs(j0)
            for s in range(_DEPTH):
                fetch(vecs0, s, s)

            @pl.loop(j0, j0 + _HALF, step=16)
            def _grp(j):
                vecs = load_vecs(j)
                nj = jnp.minimum(j + 16, _BPW - 16)
                nvecs = load_vecs(nj)
                for s in range(16):
                    slot = s % _DEPTH
                    extract(vecs, s, slot, j - half * _HALF)
                    sn = s + _DEPTH
                    if sn < 16:
                        @pl.when(j + sn < j0 + _HALF)
                        def _():
                            fetch(vecs, sn, sn % _DEPTH)
                    else:
                        @pl.when(j + sn < j0 + _HALF)
                        def _():
                            fetch(nvecs, sn - 16, sn % _DEPTH)

            for t in range(4):
                pltpu.sync_copy(stage[t],
                                outs[t].at[:, pl.ds(base + j0, _HALF)])

    return k(user, item, t0, t1, t2, t3)


_BLK = 4096


def _dense_body(umf_ref, imf_ref, umlp_ref, imlp_ref,
                w1_ref, b1_ref, w2_ref, b2_ref, wp_ref, bp_ref, o_ref):
    x = jnp.concatenate([umlp_ref[...], imlp_ref[...]], axis=0)
    h = lax.dot_general(w1_ref[...], x, (((1,), (0,)), ((), ())),
                        preferred_element_type=jnp.float32)
    h = jnp.maximum(h + b1_ref[...], 0.0)
    h = lax.dot_general(w2_ref[...], h, (((1,), (0,)), ((), ())),
                        preferred_element_type=jnp.float32)
    h = jnp.maximum(h + b2_ref[...], 0.0)
    mf = umf_ref[...] * imf_ref[...]
    s = (jnp.sum(mf * wp_ref[:_D, :], axis=0, keepdims=True)
         + jnp.sum(h * wp_ref[_D:, :], axis=0, keepdims=True)
         + bp_ref[...])
    o_ref[...] = s


def _tc_dense(umf_g, imf_g, umlp_g, imlp_g, W1, b1, W2, b2, Wp, bp):
    grid = _B // _BLK
    col = lambda i: (0, i)
    fixed2 = lambda i: (0, 0)
    out = pl.pallas_call(
        _dense_body,
        grid=(grid,),
        in_specs=[
            pl.BlockSpec((_D, _BLK), col),
            pl.BlockSpec((_D, _BLK), col),
            pl.BlockSpec((_D, _BLK), col),
            pl.BlockSpec((_D, _BLK), col),
            pl.BlockSpec((2 * _D, 2 * _D), fixed2),
            pl.BlockSpec((2 * _D, 1), fixed2),
            pl.BlockSpec((_D, 2 * _D), fixed2),
            pl.BlockSpec((_D, 1), fixed2),
            pl.BlockSpec((2 * _D, 1), fixed2),
            pl.BlockSpec((1, 1), fixed2),
        ],
        out_specs=pl.BlockSpec((1, _BLK), col),
        out_shape=jax.ShapeDtypeStruct((1, _B), jnp.float32),
    )(umf_g, imf_g, umlp_g, imlp_g, W1, b1.reshape(2 * _D, 1), W2,
      b2.reshape(_D, 1), Wp.reshape(2 * _D, 1), bp.reshape(1, 1))
    return out[0]


def kernel(user, item, user_mf, item_mf, user_mlp, item_mlp,
           W1, b1, W2, b2, Wp, bp):
    ts = [user_mf.T, item_mf.T, user_mlp.T, item_mlp.T]
    umf_g, imf_g, umlp_g, imlp_g = _sc_gather4(user, item, *ts)
    return _tc_dense(umf_g, imf_g, umlp_g, imlp_g, W1, b1, W2, b2, Wp, bp)
